# baseline (speedup 1.0000x reference)
import functools

import jax
import jax.numpy as jnp
from jax import lax
from jax.experimental import pallas as pl
from jax.experimental.pallas import tpu as pltpu
from jax.experimental.pallas import tpu_sc as plsc

_BATCH = 16384
_DIM = 32
_K = 3 * _BATCH
_NW = 32
_BPW = _BATCH // _NW
_WL = 512
_BLKS_PER_TILE = 61
_NWIN = 62
_TAIL_BASE = 999936
_CAP = 8192
_STAGE = 2048
_SCRATCH = _DIM * _K
_SCRATCH_PAD = _SCRATCH + _NW * _STAGE

_mesh = plsc.VectorSubcoreMesh(core_axis_name="c", subcore_axis_name="s")
_IOTA = lambda: lax.iota(jnp.int32, 16)


@functools.partial(
    pl.kernel,
    mesh=_mesh,
    out_type=jax.ShapeDtypeStruct((_SCRATCH_PAD,), jnp.float32),
    compiler_params=pltpu.CompilerParams(
        needs_layout_passes=False, use_tc_tiling_on_sc=True),
    scratch_types=[
        pltpu.VMEM((1024,), jnp.int32),
        pltpu.VMEM((1024,), jnp.int32),
        pltpu.VMEM((_CAP + 16,), jnp.int32),
        pltpu.VMEM((_CAP + 16,), jnp.int32),
        pltpu.VMEM((_CAP + 16,), jnp.int32),
        pltpu.VMEM((_CAP + 16,), jnp.int32),
        pltpu.VMEM((_DIM, _WL), jnp.float32),
        pltpu.VMEM((_DIM, _WL), jnp.float32),
        pltpu.VMEM((_DIM, _WL), jnp.float32),
        pltpu.VMEM((_DIM, _WL), jnp.float32),
        pltpu.VMEM((_DIM, 64), jnp.float32),
        pltpu.VMEM((_DIM, 64), jnp.float32),
        pltpu.VMEM((16,), jnp.int32),
        pltpu.VMEM((16,), jnp.int32),
        pltpu.VMEM((_STAGE,), jnp.float32),
        pltpu.VMEM((_STAGE,), jnp.int32),
        pltpu.VMEM((_STAGE,), jnp.float32),
        pltpu.VMEM((_STAGE,), jnp.int32),
        pltpu.SemaphoreType.DMA,
        pltpu.SemaphoreType.DMA,
        pltpu.SemaphoreType.DMA,
        pltpu.SemaphoreType.DMA,
        pltpu.SemaphoreType.DMA,
        pltpu.SemaphoreType.DMA,
    ],
)
def _harvest(tuples_hbm, nodeT_hbm, relaT_hbm, scratch_hbm,
             cbuf0, cbuf1, hNi, hNk, hRi, hRk,
             nwin0, nwin1, rwin0, rwin1, ntail, rtail,
             clane, cslot, stv0, std0, stv1, std1,
             semi0, semi1, semw0, semw1, semf0, semf1):
    wid = lax.axis_index("s") * 2 + lax.axis_index("c")
    blk0 = wid * _BLKS_PER_TILE
    lane_lo = blk0 * _WL
    is_last = wid == _NW - 1
    lane_hi = jnp.where(is_last, 1000000, lane_lo + _BLKS_PER_TILE * _WL)
    iota = _IOTA()
    dump_base = _SCRATCH + wid * _STAGE

    cbufs = (cbuf0, cbuf1)
    nwins = (nwin0, nwin1)
    rwins = (rwin0, rwin1)
    semis = (semi0, semi1)
    semws = (semw0, semw1)
    stvs = (stv0, stv1)
    stds = (std0, std1)
    semfs = (semf0, semf1)
    NCHUNK = _K // 1024
    CAPW = _STAGE - 32

    def reset_dump(std):
        for s in range(_STAGE // 16):
            std[pl.ds(s * 16, 16)] = dump_base + s * 16 + iota

    for b in range(2):
        reset_dump(stds[b])
        pltpu.async_copy(stvs[b], scratch_hbm.at[stds[b]], semfs[b])

    def scan_round(rlo):
        pltpu.async_copy(tuples_hbm.at[pl.ds(0, 1024)], cbufs[0], semis[0])
        pltpu.async_copy(tuples_hbm.at[pl.ds(1024, 1024)], cbufs[1], semis[1])

        def pair_body(p, carry):
            cN, cR = carry
            for b in range(2):
                c = 2 * p + b
                pltpu.make_async_copy(
                    tuples_hbm.at[pl.ds(0, 1024)], cbufs[b], semis[b]).wait()

                def vreg_body(v, carry2, b=b, c=c):
                    cN2, cR2 = carry2
                    iv = cbufs[b][pl.ds(v * 16, 16)]
                    kvec = c * 1024 + v * 16 + iota
                    inr = (iv >= lane_lo) & (iv < lane_hi)
                    mN = inr & (kvec < 2 * _BATCH)
                    mR = inr & (kvec >= 2 * _BATCH)

                    def emit(mask, cnt, hi_ref, hk_ref):
                        mi = mask.astype(jnp.int32)
                        csum = lax.cumsum(mi, axis=0)
                        rank = cnt + csum - 1
                        ok = mask & (rank >= rlo) & (rank < rlo + _CAP)
                        pos = jnp.clip(cnt - rlo, 0, _CAP)
                        plsc.store_compressed(hi_ref.at[pl.ds(pos, 16)], iv,
                                              mask=ok)
                        plsc.store_compressed(hk_ref.at[pl.ds(pos, 16)],
                                              kvec, mask=ok)
                        return cnt + csum[15]

                    cN2 = emit(mN, cN2, hNi, hNk)
                    cR2 = emit(mR, cR2, hRi, hRk)
                    return cN2, cR2

                cN, cR = lax.fori_loop(0, 64, vreg_body, (cN, cR))

                def refire(c=c, b=b):
                    off = pl.multiple_of((c + 2) * 1024, 1024)
                    pltpu.async_copy(
                        tuples_hbm.at[pl.ds(off, 1024)], cbufs[b], semis[b])

                pl.when(c + 2 < NCHUNK)(refire)
            return cN, cR

        return lax.fori_loop(0, NCHUNK // 2, pair_body,
                             (jnp.int32(0), jnp.int32(0)))

    def harvest_win(buf, width, wbase, nhits, hi_ref, hk_ref, carry,
                    stv, std, sbase):
        nv = (nhits + 15) // 16

        def vreg_body(v, carry2):
            c2, skipped = carry2
            hv = hi_ref[pl.ds(v * 16, 16)]
            hk = hk_ref[pl.ds(v * 16, 16)]
            lanev = hv - wbase
            inwin = (hv >= wbase) & (hv < wbase + width)
            inwin = inwin & ((v * 16 + iota) < nhits)
            m = jnp.sum(inwin.astype(jnp.int32))
            plsc.store_compressed(clane.at[pl.ds(0, 16)], lanev, mask=inwin)
            plsc.store_compressed(cslot.at[pl.ds(0, 16)], hk, mask=inwin)

            def hit_body(j, carry3):
                c2, skipped = carry3
                jv = jnp.full((16,), j, jnp.int32)
                colv = plsc.load_gather(clane, [jv])
                kv = plsc.load_gather(cslot, [jv])
                v_lo = plsc.load_gather(buf, [iota, colv])
                v_hi = plsc.load_gather(buf, [iota + 16, colv])
                d_lo = kv * _DIM + iota
                d_hi = kv * _DIM + 16 + iota
                in_sweep = (c2 >= sbase) & (c2 < sbase + CAPW)
                pos = jnp.where(in_sweep, c2 - sbase, CAPW)
                stv[pl.ds(pos, 16)] = v_lo
                std[pl.ds(pos, 16)] = d_lo
                stv[pl.ds(pos + 16, 16)] = v_hi
                std[pl.ds(pos + 16, 16)] = d_hi
                skipped = skipped + jnp.where(c2 >= sbase + CAPW, 1, 0)
                return c2 + 32, skipped

            return lax.fori_loop(0, m, hit_body, (c2, skipped))

        return lax.fori_loop(0, nv, vreg_body, carry)

    def run_rounds(sbase):
        def round_body(carry):
            r, skipped, doneN, doneR, totN, totR = carry
            rlo = r * _CAP
            totN, totR = scan_round(rlo)
            nN = jnp.clip(totN - rlo, 0, _CAP)
            nR = jnp.clip(totR - rlo, 0, _CAP)

            off0 = pl.multiple_of(blk0 * _WL, _WL)
            off1 = pl.multiple_of((blk0 + 1) * _WL, _WL)
            pltpu.async_copy(nodeT_hbm.at[:, pl.ds(off0, _WL)], nwins[0],
                             semws[0])
            pltpu.async_copy(relaT_hbm.at[:, pl.ds(off0, _WL)], rwins[0],
                             semws[0])
            pltpu.async_copy(nodeT_hbm.at[:, pl.ds(off1, _WL)], nwins[1],
                             semws[1])
            pltpu.async_copy(relaT_hbm.at[:, pl.ds(off1, _WL)], rwins[1],
                             semws[1])

            def win_pair(p, skipped):
                for b in range(2):
                    w = 2 * p + b
                    pltpu.make_async_copy(
                        nodeT_hbm.at[:, pl.ds(0, _WL)], nwins[b],
                        semws[b]).wait()
                    pltpu.make_async_copy(
                        relaT_hbm.at[:, pl.ds(0, _WL)], rwins[b],
                        semws[b]).wait()
                    pltpu.make_async_copy(
                        stvs[b], scratch_hbm.at[stds[b]],
                        semfs[b]).wait()
                    reset_dump(stds[b])
                    wbase = (blk0 + w) * _WL
                    c2 = jnp.int32(0)
                    c2, skipped = harvest_win(
                        nwins[b], _WL, wbase, nN, hNi, hNk, (c2, skipped),
                        stvs[b], stds[b], sbase)
                    c2, skipped = harvest_win(
                        rwins[b], _WL, wbase, nR, hRi, hRk, (c2, skipped),
                        stvs[b], stds[b], sbase)
                    pltpu.async_copy(stvs[b], scratch_hbm.at[stds[b]],
                                     semfs[b])

                    def refire(w=w, b=b):
                        off = pl.multiple_of((blk0 + w + 2) * _WL, _WL)
                        pltpu.async_copy(nodeT_hbm.at[:, pl.ds(off, _WL)],
                                         nwins[b], semws[b])
                        pltpu.async_copy(relaT_hbm.at[:, pl.ds(off, _WL)],
                                         rwins[b], semws[b])

                    pl.when(w + 2 < _NWIN)(refire)
                return skipped

            skipped = lax.fori_loop(0, _NWIN // 2, win_pair, skipped)

            def tail(skipped):
                pltpu.async_copy(nodeT_hbm.at[:, pl.ds(_TAIL_BASE, 64)],
                                 ntail, semws[0]).wait()
                pltpu.async_copy(relaT_hbm.at[:, pl.ds(_TAIL_BASE, 64)],
                                 rtail, semws[0]).wait()
                pltpu.make_async_copy(
                    stvs[0], scratch_hbm.at[stds[0]], semfs[0]).wait()
                reset_dump(stds[0])
                c2 = jnp.int32(0)
                c2, skipped = harvest_win(
                    ntail, 64, _TAIL_BASE, nN, hNi, hNk, (c2, skipped),
                    stvs[0], stds[0], sbase)
                c2, skipped = harvest_win(
                    rtail, 64, _TAIL_BASE, nR, hRi, hRk, (c2, skipped),
                    stvs[0], stds[0], sbase)
                pltpu.async_copy(stvs[0], scratch_hbm.at[stds[0]],
                                 semfs[0])
                return skipped

            skipped = lax.cond(is_last, tail, lambda s: s, skipped)
            doneN = jnp.minimum(rlo + _CAP, totN)
            doneR = jnp.minimum(rlo + _CAP, totR)
            return r + 1, skipped, doneN, doneR, totN, totR

        def round_cond(carry):
            r, skipped, doneN, doneR, totN, totR = carry
            return (r == 0) | (doneN < totN) | (doneR < totR)

        init = (jnp.int32(0), jnp.int32(0), jnp.int32(0), jnp.int32(0),
                jnp.int32(-1), jnp.int32(-1))
        _, skipped, _, _, _, _ = lax.while_loop(round_cond, round_body, init)
        return skipped

    def sweep_cond(carry):
        s, skipped = carry
        return (s == 0) | (skipped > 0)

    def sweep_body(carry):
        s, _ = carry
        skipped = run_rounds(s * CAPW)
        return s + 1, skipped

    lax.while_loop(sweep_cond, sweep_body, (jnp.int32(0), jnp.int32(0)))

    for b in range(2):
        pltpu.make_async_copy(stvs[b], scratch_hbm.at[stds[b]],
                              semfs[b]).wait()


@functools.partial(
    pl.kernel,
    mesh=_mesh,
    out_type=jax.ShapeDtypeStruct((_BATCH,), jnp.float32),
    compiler_params=pltpu.CompilerParams(
        needs_layout_passes=False, use_tc_tiling_on_sc=False),
    scratch_types=[
        pltpu.VMEM((_BPW, _DIM), jnp.float32),
        pltpu.VMEM((_BPW, _DIM), jnp.float32),
        pltpu.VMEM((_BPW, _DIM), jnp.float32),
        pltpu.VMEM((_BPW,), jnp.float32),
        pltpu.SemaphoreType.DMA,
    ],
)
def _reduce(scratch_hbm, out_hbm, hrows, trows, rrows, outv, sem):
    wid = lax.axis_index("s") * 2 + lax.axis_index("c")
    base = wid * _BPW
    h1 = pltpu.async_copy(scratch_hbm.at[pl.ds(base, _BPW), :], hrows, sem)
    h2 = pltpu.async_copy(
        scratch_hbm.at[pl.ds(_BATCH + base, _BPW), :], trows, sem)
    h3 = pltpu.async_copy(
        scratch_hbm.at[pl.ds(2 * _BATCH + base, _BPW), :], rrows, sem)
    h1.wait()
    h2.wait()
    h3.wait()

    def chunk_body(c, carry):
        rows = c * 16 + lax.iota(jnp.int32, 16)

        def d_body(d, acc):
            cols = jnp.full((16,), d, jnp.int32)
            hv = plsc.load_gather(hrows, [rows, cols])
            tv = plsc.load_gather(trows, [rows, cols])
            rv = plsc.load_gather(rrows, [rows, cols])
            return acc + hv * rv * tv

        acc = lax.fori_loop(0, _DIM, d_body, jnp.zeros((16,), jnp.float32))
        outv[pl.ds(c * 16, 16)] = acc
        return carry

    lax.fori_loop(0, _BPW // 16, chunk_body, 0)
    pltpu.sync_copy(outv, out_hbm.at[pl.ds(base, _BPW)])


def kernel(tuples, node_emb, rela_emb):
    scratch = _harvest(tuples.reshape(-1), node_emb.T, rela_emb.T)
    return _reduce(scratch.reshape(_SCRATCH_PAD // _DIM, _DIM))


# speedup vs baseline: 10.1826x; 10.1826x over previous
import functools

import jax
import jax.numpy as jnp
from jax import lax
from jax.experimental import pallas as pl
from jax.experimental.pallas import tpu as pltpu
from jax.experimental.pallas import tpu_sc as plsc

_BATCH = 16384
_DIM = 32
_NUM_CORES = 2
_NUM_SUBCORES = 16
_NW = _NUM_CORES * _NUM_SUBCORES
_BPW = _BATCH // _NW
_IDX_CHUNK = 128
_NCHUNK = _BPW // _IDX_CHUNK

_mesh = plsc.VectorSubcoreMesh(core_axis_name="c", subcore_axis_name="s")


@functools.partial(
    pl.kernel,
    mesh=_mesh,
    out_type=jax.ShapeDtypeStruct((_BATCH,), jnp.float32),
    compiler_params=pltpu.CompilerParams(
        needs_layout_passes=False, use_tc_tiling_on_sc=False),
    scratch_types=[
        pltpu.VMEM((_BPW,), jnp.int32),
        pltpu.VMEM((_BPW,), jnp.int32),
        pltpu.VMEM((_BPW,), jnp.int32),
        pltpu.VMEM((_BPW, _DIM), jnp.float32),
        pltpu.VMEM((_BPW, _DIM), jnp.float32),
        pltpu.VMEM((_BPW, _DIM), jnp.float32),
        pltpu.VMEM((_BPW,), jnp.float32),
        pltpu.SemaphoreType.DMA,
    ],
)
def _distmult_sc(tuples_hbm, node_hbm, rela_hbm, out_hbm,
                 hidx, tidx, ridx, hrows, trows, rrows, outv, sem):
    wid = lax.axis_index("s") * _NUM_CORES + lax.axis_index("c")
    base = wid * _BPW

    pltpu.sync_copy(tuples_hbm.at[pl.ds(base, _BPW)], hidx)
    pltpu.sync_copy(tuples_hbm.at[pl.ds(_BATCH + base, _BPW)], tidx)
    pltpu.sync_copy(tuples_hbm.at[pl.ds(2 * _BATCH + base, _BPW)], ridx)

    copies = []
    for j in range(_NCHUNK):
        s = pl.ds(j * _IDX_CHUNK, _IDX_CHUNK)
        copies.append(pltpu.async_copy(node_hbm.at[hidx.at[s]], hrows.at[s], sem))
        copies.append(pltpu.async_copy(node_hbm.at[tidx.at[s]], trows.at[s], sem))
        copies.append(pltpu.async_copy(rela_hbm.at[ridx.at[s]], rrows.at[s], sem))
    for c in copies:
        c.wait()

    def chunk_body(c, carry):
        rows = c * 16 + lax.iota(jnp.int32, 16)

        def d_body(d, acc):
            cols = jnp.full((16,), d, jnp.int32)
            hv = plsc.load_gather(hrows, [rows, cols])
            tv = plsc.load_gather(trows, [rows, cols])
            rv = plsc.load_gather(rrows, [rows, cols])
            return acc + hv * rv * tv

        acc = lax.fori_loop(0, _DIM, d_body, jnp.zeros((16,), jnp.float32))
        outv[pl.ds(c * 16, 16)] = acc
        return carry

    lax.fori_loop(0, _BPW // 16, chunk_body, 0)

    pltpu.sync_copy(outv, out_hbm.at[pl.ds(base, _BPW)])


def kernel(tuples, node_emb, rela_emb):
    return _distmult_sc(tuples.reshape(-1), node_emb, rela_emb)
